# baseline (speedup 1.0000x reference)
import functools

import jax
import jax.numpy as jnp
from jax import lax
from jax.experimental import pallas as pl
from jax.experimental.pallas import tpu as pltpu
from jax.experimental.pallas import tpu_sc as plsc

B = 16384
F = 64
NC = 2
NS = 16
NW = NC * NS
BPW = B // NW
CH = 128
NCH = BPW // CH
L = 16


def _body(user_hbm, item_hbm, uf_hbm, if_hbm, ub_hbm, ib_hbm, out_hbm,
          uidx, iidx, urows, irows, ubias, ibias, outv, sem):
    c = lax.axis_index("c")
    s = lax.axis_index("s")
    wid = s * NC + c
    row0 = wid * NCH

    pltpu.sync_copy(user_hbm.at[pl.ds(row0, NCH)], uidx)
    pltpu.sync_copy(item_hbm.at[pl.ds(row0, NCH)], iidx)

    copies = []
    for k in range(NCH):
        dst = pl.ds(k * CH, CH)
        copies.append(pltpu.async_copy(uf_hbm.at[uidx.at[k]], urows.at[dst], sem))
        copies.append(pltpu.async_copy(if_hbm.at[iidx.at[k]], irows.at[dst], sem))
        copies.append(pltpu.async_copy(ub_hbm.at[uidx.at[k]], ubias.at[dst], sem))
        copies.append(pltpu.async_copy(ib_hbm.at[iidx.at[k]], ibias.at[dst], sem))
    for cp in copies:
        cp.wait()

    lane = lax.iota(jnp.int32, L)

    def grp(g, carry):
        base = g * L
        res = jnp.zeros((L,), jnp.float32)
        for t in range(L):
            r = base + t
            acc = urows[r, pl.ds(0, L)] * irows[r, pl.ds(0, L)]
            for q in range(1, F // L):
                acc = acc + urows[r, pl.ds(q * L, L)] * irows[r, pl.ds(q * L, L)]
            res = jnp.where(lane == t, jnp.sum(acc), res)
        sl = pl.ds(base, L)
        outv[sl] = res + ubias[sl] + ibias[sl]
        return carry

    lax.fori_loop(0, BPW // L, grp, None)

    pltpu.sync_copy(outv, out_hbm.at[pl.ds(wid * BPW, BPW)])


@jax.jit
def _sc_call(user2, item2, uf, itf, ub, ib):
    grid_kernel = functools.partial(
        pl.kernel,
        out_type=jax.ShapeDtypeStruct((B,), jnp.float32),
        mesh=plsc.VectorSubcoreMesh(core_axis_name="c", subcore_axis_name="s"),
        compiler_params=pltpu.CompilerParams(
            needs_layout_passes=False, use_tc_tiling_on_sc=False),
        scratch_types=[
            pltpu.VMEM((NCH, CH), jnp.int32),
            pltpu.VMEM((NCH, CH), jnp.int32),
            pltpu.VMEM((BPW, F), jnp.float32),
            pltpu.VMEM((BPW, F), jnp.float32),
            pltpu.VMEM((BPW,), jnp.float32),
            pltpu.VMEM((BPW,), jnp.float32),
            pltpu.VMEM((BPW,), jnp.float32),
            pltpu.SemaphoreType.DMA,
        ],
    )
    return grid_kernel(_body)(user2, item2, uf, itf, ub, ib)


def kernel(user, item, user_factors, item_factors, users_biases, items_biases):
    user2 = user.astype(jnp.int32).reshape(B // CH, CH)
    item2 = item.astype(jnp.int32).reshape(B // CH, CH)
    ub = users_biases.reshape(-1)
    ib = items_biases.reshape(-1)
    return _sc_call(user2, item2, user_factors, item_factors, ub, ib)
